# baseline (speedup 1.0000x reference)
import functools

import jax
import jax.numpy as jnp
from jax import lax
from jax.experimental import pallas as pl
from jax.experimental.pallas import tpu as pltpu
from jax.experimental.pallas import tpu_sc as plsc

T = 8
V = 1000
N = 1000
NPAD = 1008
ROWS = 16384 * T
NC, NS = 2, 16
NW = NC * NS
BPW = ROWS // NW
CHUNK = 64


def _table_body(tok_ref, pos_ref, w_ref, b_ref, out_ref):
    tl = jnp.dot(tok_ref[...], w_ref[...], preferred_element_type=jnp.float32)
    pp = jnp.dot(pos_ref[...], w_ref[...], preferred_element_type=jnp.float32)
    pp = pp + b_ref[...]
    out_ref[...] = tl[:, None, :] + pp[None, :, :]


def _build_table(tok, pos, w_pad, b_pad):
    vb = 200
    return pl.pallas_call(
        _table_body,
        grid=(V // vb,),
        in_specs=[
            pl.BlockSpec((vb, 64), lambda i: (i, 0)),
            pl.BlockSpec((T, 64), lambda i: (0, 0)),
            pl.BlockSpec((64, NPAD), lambda i: (0, 0)),
            pl.BlockSpec((1, NPAD), lambda i: (0, 0)),
        ],
        out_specs=pl.BlockSpec((vb, T, NPAD), lambda i: (i, 0, 0)),
        out_shape=jax.ShapeDtypeStruct((V, T, NPAD), jnp.float32),
    )(tok, pos, w_pad, b_pad)


def _gather_body(table_hbm, idx_hbm, out_hbm, idx_v, cidx_v, rows_v, sem):
    wid = lax.axis_index("s") * NC + lax.axis_index("c")
    base = wid * BPW
    pltpu.sync_copy(idx_hbm.at[pl.ds(base, BPW)], idx_v)

    tpat = lax.iota(jnp.int32, 16) & 7

    def cbody(i, carry):
        s = pl.ds(i * 16, 16)
        cidx_v[s] = idx_v[s] * 8 + tpat
        return carry

    lax.fori_loop(0, BPW // 16, cbody, 0)

    def gbody(g, carry):
        off = g * CHUNK
        pltpu.async_copy(
            table_hbm.at[cidx_v.at[pl.ds(off, CHUNK)]], rows_v, sem
        ).wait()
        pltpu.sync_copy(
            rows_v.at[:, pl.ds(0, N)],
            out_hbm.at[pl.ds(base + off, CHUNK), :],
        )
        return carry

    lax.fori_loop(0, BPW // CHUNK, gbody, 0)


@functools.partial(
    pl.kernel,
    mesh=plsc.VectorSubcoreMesh(core_axis_name="c", subcore_axis_name="s"),
    out_type=jax.ShapeDtypeStruct((ROWS, N), jnp.float32),
    compiler_params=pltpu.CompilerParams(use_tc_tiling_on_sc=False),
    scratch_types=[
        pltpu.VMEM((BPW,), jnp.int32),
        pltpu.VMEM((BPW,), jnp.int32),
        pltpu.VMEM((CHUNK, NPAD), jnp.float32),
        pltpu.SemaphoreType.DMA,
    ],
)
def _gather(table_hbm, idx_hbm, out_hbm, idx_v, cidx_v, rows_v, sem):
    _gather_body(table_hbm, idx_hbm, out_hbm, idx_v, cidx_v, rows_v, sem)


def kernel(idx, token_embed_table, pos_embed_table, lm_head_w, lm_head_b):
    idx = idx[:, -T:]
    w_pad = jnp.pad(lm_head_w, ((0, 0), (0, NPAD - N)))
    b_pad = jnp.pad(lm_head_b, (0, NPAD - N)).reshape(1, NPAD)
    table = _build_table(token_embed_table, pos_embed_table, w_pad, b_pad)
    table = table.reshape(V * T, NPAD)
    idx_flat = idx.reshape(ROWS).astype(jnp.int32)
    out = _gather(table, idx_flat)
    return out.reshape(idx.shape[0], T, N)


# speedup vs baseline: 1.0198x; 1.0198x over previous
import functools

import jax
import jax.numpy as jnp
from jax import lax
from jax.experimental import pallas as pl
from jax.experimental.pallas import tpu as pltpu
from jax.experimental.pallas import tpu_sc as plsc

T = 8
V = 1000
N = 1000
NPAD = 1008
ROWS = 16384 * T
NC, NS = 2, 16
NW = NC * NS
BPW = ROWS // NW
CHUNK = 32
NCH = BPW // CHUNK


def _table_body(tok_ref, pos_ref, w_ref, b_ref, out_ref):
    tl = jnp.dot(tok_ref[...], w_ref[...], preferred_element_type=jnp.float32)
    pp = jnp.dot(pos_ref[...], w_ref[...], preferred_element_type=jnp.float32)
    pp = pp + b_ref[...]
    out_ref[...] = tl[:, None, :] + pp[None, :, :]


def _build_table(tok, pos, w_pad, b_pad):
    vb = 200
    return pl.pallas_call(
        _table_body,
        grid=(V // vb,),
        in_specs=[
            pl.BlockSpec((vb, 64), lambda i: (i, 0)),
            pl.BlockSpec((T, 64), lambda i: (0, 0)),
            pl.BlockSpec((64, NPAD), lambda i: (0, 0)),
            pl.BlockSpec((1, NPAD), lambda i: (0, 0)),
        ],
        out_specs=pl.BlockSpec((vb, T, NPAD), lambda i: (i, 0, 0)),
        out_shape=jax.ShapeDtypeStruct((V, T, NPAD), jnp.float32),
    )(tok, pos, w_pad, b_pad)


def _gather_body(table_hbm, idx_hbm, out_hbm, idx_v, cidx_v,
                 buf0, buf1, semg0, semg1, semw0, semw1):
    wid = lax.axis_index("s") * NC + lax.axis_index("c")
    base = wid * BPW
    pltpu.sync_copy(idx_hbm.at[pl.ds(base, BPW)], idx_v)

    tpat = lax.iota(jnp.int32, 16) & 7

    def cbody(i, carry):
        s = pl.ds(i * 16, 16)
        cidx_v[s] = idx_v[s] * 8 + tpat
        return carry

    lax.fori_loop(0, BPW // 16, cbody, 0)

    bufs = (buf0, buf1)
    semgs = (semg0, semg1)
    semws = (semw0, semw1)

    def start_gather(g, b):
        pltpu.async_copy(
            table_hbm.at[cidx_v.at[pl.ds(g * CHUNK, CHUNK)]], bufs[b], semgs[b]
        )

    def wait_gather(b):
        pltpu.make_async_copy(
            table_hbm.at[cidx_v.at[pl.ds(0, CHUNK)]], bufs[b], semgs[b]
        ).wait()

    def start_write(g, b):
        pltpu.async_copy(
            bufs[b].at[:, pl.ds(0, N)],
            out_hbm.at[pl.ds(base + g * CHUNK, CHUNK), :],
            semws[b],
        )

    def wait_write(b):
        pltpu.make_async_copy(
            bufs[b].at[:, pl.ds(0, N)],
            out_hbm.at[pl.ds(base, CHUNK), :],
            semws[b],
        ).wait()

    start_gather(0, 0)
    wait_gather(0)
    start_gather(1, 1)
    start_write(0, 0)

    def lbody(k, carry):
        g_odd = 2 * k + 1
        wait_write(0)
        start_gather(g_odd + 1, 0)
        wait_gather(1)
        start_write(g_odd, 1)
        g_even = 2 * k + 2
        wait_write(1)
        start_gather(g_even + 1, 1)
        wait_gather(0)
        start_write(g_even, 0)
        return carry

    lax.fori_loop(0, NCH // 2 - 1, lbody, 0)

    wait_write(0)
    wait_gather(1)
    start_write(NCH - 1, 1)
    wait_write(1)


@functools.partial(
    pl.kernel,
    mesh=plsc.VectorSubcoreMesh(core_axis_name="c", subcore_axis_name="s"),
    out_type=jax.ShapeDtypeStruct((ROWS, N), jnp.float32),
    compiler_params=pltpu.CompilerParams(use_tc_tiling_on_sc=False),
    scratch_types=[
        pltpu.VMEM((BPW,), jnp.int32),
        pltpu.VMEM((BPW,), jnp.int32),
        pltpu.VMEM((CHUNK, NPAD), jnp.float32),
        pltpu.VMEM((CHUNK, NPAD), jnp.float32),
        pltpu.SemaphoreType.DMA,
        pltpu.SemaphoreType.DMA,
        pltpu.SemaphoreType.DMA,
        pltpu.SemaphoreType.DMA,
    ],
)
def _gather(table_hbm, idx_hbm, out_hbm, idx_v, cidx_v,
            buf0, buf1, semg0, semg1, semw0, semw1):
    _gather_body(table_hbm, idx_hbm, out_hbm, idx_v, cidx_v,
                 buf0, buf1, semg0, semg1, semw0, semw1)


def kernel(idx, token_embed_table, pos_embed_table, lm_head_w, lm_head_b):
    idx = idx[:, -T:]
    w_pad = jnp.pad(lm_head_w, ((0, 0), (0, NPAD - N)))
    b_pad = jnp.pad(lm_head_b, (0, NPAD - N)).reshape(1, NPAD)
    table = _build_table(token_embed_table, pos_embed_table, w_pad, b_pad)
    table = table.reshape(V * T, NPAD)
    idx_flat = idx.reshape(ROWS).astype(jnp.int32)
    out = _gather(table, idx_flat)
    return out.reshape(idx.shape[0], T, N)


# speedup vs baseline: 3.3334x; 3.2688x over previous
import functools

import jax
import jax.numpy as jnp
from jax import lax
from jax.experimental import pallas as pl
from jax.experimental.pallas import tpu as pltpu
from jax.experimental.pallas import tpu_sc as plsc

T = 8
V = 1000
N = 1000
D = 64
B = 16384
ROWS = B * T
NC, NS = 2, 16
NW = NC * NS
BPW = ROWS // NW
CHUNK = 128
NCH = BPW // CHUNK
BB = 512


def _gather_body(tok_hbm, idx_hbm, out_hbm, idx_v, buf0, buf1,
                 semg0, semg1, semw0, semw1):
    wid = lax.axis_index("s") * NC + lax.axis_index("c")
    base = wid * BPW
    pltpu.sync_copy(idx_hbm.at[pl.ds(base, BPW)], idx_v)

    bufs = (buf0, buf1)
    semgs = (semg0, semg1)
    semws = (semw0, semw1)

    def start_gather(g, b):
        pltpu.async_copy(
            tok_hbm.at[idx_v.at[pl.ds(g * CHUNK, CHUNK)]], bufs[b], semgs[b]
        )

    def wait_gather(b):
        pltpu.make_async_copy(
            tok_hbm.at[idx_v.at[pl.ds(0, CHUNK)]], bufs[b], semgs[b]
        ).wait()

    def start_write(g, b):
        pltpu.async_copy(
            bufs[b], out_hbm.at[pl.ds(base + g * CHUNK, CHUNK), :], semws[b]
        )

    def wait_write(b):
        pltpu.make_async_copy(
            bufs[b], out_hbm.at[pl.ds(base, CHUNK), :], semws[b]
        ).wait()

    start_gather(0, 0)
    wait_gather(0)
    start_gather(1, 1)
    start_write(0, 0)

    def lbody(k, carry):
        g_odd = 2 * k + 1
        wait_write(0)
        start_gather(g_odd + 1, 0)
        wait_gather(1)
        start_write(g_odd, 1)
        g_even = 2 * k + 2
        wait_write(1)
        start_gather(g_even + 1, 1)
        wait_gather(0)
        start_write(g_even, 0)
        return carry

    lax.fori_loop(0, NCH // 2 - 1, lbody, 0)

    wait_write(0)
    wait_gather(1)
    start_write(NCH - 1, 1)
    wait_write(1)


@functools.partial(
    pl.kernel,
    mesh=plsc.VectorSubcoreMesh(core_axis_name="c", subcore_axis_name="s"),
    out_type=jax.ShapeDtypeStruct((ROWS, D), jnp.float32),
    compiler_params=pltpu.CompilerParams(use_tc_tiling_on_sc=False),
    scratch_types=[
        pltpu.VMEM((BPW,), jnp.int32),
        pltpu.VMEM((CHUNK, D), jnp.float32),
        pltpu.VMEM((CHUNK, D), jnp.float32),
        pltpu.SemaphoreType.DMA,
        pltpu.SemaphoreType.DMA,
        pltpu.SemaphoreType.DMA,
        pltpu.SemaphoreType.DMA,
    ],
)
def _gather(tok_hbm, idx_hbm, out_hbm, idx_v, buf0, buf1,
            semg0, semg1, semw0, semw1):
    _gather_body(tok_hbm, idx_hbm, out_hbm, idx_v, buf0, buf1,
                 semg0, semg1, semw0, semw1)


def _proj_body(emb_ref, w_ref, pos_ref, b_ref, out_ref):
    t = pl.program_id(0)
    e_bf = emb_ref[0].astype(jnp.bfloat16)
    w_bf = w_ref[...].astype(jnp.bfloat16)
    pos_bf = pos_ref[pl.ds(t, 1), :].astype(jnp.bfloat16)
    r = lax.dot_general(w_bf, e_bf, (((0,), (1,)), ((), ())),
                        preferred_element_type=jnp.float32)
    r2 = lax.dot_general(w_bf, pos_bf, (((0,), (1,)), ((), ())),
                         preferred_element_type=jnp.float32)
    out_ref[0] = r + r2 + b_ref[...]


def _project(emb3, w, pos, b_col):
    return pl.pallas_call(
        _proj_body,
        grid=(T, B // BB),
        in_specs=[
            pl.BlockSpec((1, BB, D), lambda t, j: (t, j, 0)),
            pl.BlockSpec((D, N), lambda t, j: (0, 0)),
            pl.BlockSpec((T, D), lambda t, j: (0, 0)),
            pl.BlockSpec((N, 1), lambda t, j: (0, 0)),
        ],
        out_specs=pl.BlockSpec((1, N, BB), lambda t, j: (t, 0, j)),
        out_shape=jax.ShapeDtypeStruct((T, N, B), jnp.float32),
    )(emb3, w, pos, b_col)


def kernel(idx, token_embed_table, pos_embed_table, lm_head_w, lm_head_b):
    idx = idx[:, -T:]
    idx_t = jnp.transpose(idx).reshape(ROWS).astype(jnp.int32)
    emb = _gather(token_embed_table, idx_t)
    emb3 = emb.reshape(T, B, D)
    out_t = _project(emb3, lm_head_w, pos_embed_table,
                     lm_head_b.reshape(N, 1))
    return jnp.transpose(out_t, (2, 0, 1))


# speedup vs baseline: 4.1005x; 1.2301x over previous
import functools

import jax
import jax.numpy as jnp
from jax import lax
from jax.experimental import pallas as pl
from jax.experimental.pallas import tpu as pltpu
from jax.experimental.pallas import tpu_sc as plsc

T = 8
V = 1000
N = 1000
D = 64
B = 16384
ROWS = B * T
NC, NS = 2, 16
NW = NC * NS
BPW = ROWS // NW
CHUNK = 128
NCH = BPW // CHUNK
BB = 1024


def _gather_body(tok_hbm, idx_hbm, out_hbm, idx_v, buf0, buf1,
                 semg0, semg1, semw0, semw1):
    wid = lax.axis_index("s") * NC + lax.axis_index("c")
    base = wid * BPW
    pltpu.sync_copy(idx_hbm.at[pl.ds(base, BPW)], idx_v)

    bufs = (buf0, buf1)
    semgs = (semg0, semg1)
    semws = (semw0, semw1)

    def start_gather(g, b):
        pltpu.async_copy(
            tok_hbm.at[idx_v.at[pl.ds(g * CHUNK, CHUNK)]], bufs[b], semgs[b]
        )

    def wait_gather(b):
        pltpu.make_async_copy(
            tok_hbm.at[idx_v.at[pl.ds(0, CHUNK)]], bufs[b], semgs[b]
        ).wait()

    def start_write(g, b):
        pltpu.async_copy(
            bufs[b], out_hbm.at[pl.ds(base + g * CHUNK, CHUNK), :], semws[b]
        )

    def wait_write(b):
        pltpu.make_async_copy(
            bufs[b], out_hbm.at[pl.ds(base, CHUNK), :], semws[b]
        ).wait()

    start_gather(0, 0)
    wait_gather(0)
    start_gather(1, 1)
    start_write(0, 0)

    def lbody(k, carry):
        g_odd = 2 * k + 1
        wait_write(0)
        start_gather(g_odd + 1, 0)
        wait_gather(1)
        start_write(g_odd, 1)
        g_even = 2 * k + 2
        wait_write(1)
        start_gather(g_even + 1, 1)
        wait_gather(0)
        start_write(g_even, 0)
        return carry

    lax.fori_loop(0, NCH // 2 - 1, lbody, 0)

    wait_write(0)
    wait_gather(1)
    start_write(NCH - 1, 1)
    wait_write(1)


@functools.partial(
    pl.kernel,
    mesh=plsc.VectorSubcoreMesh(core_axis_name="c", subcore_axis_name="s"),
    out_type=jax.ShapeDtypeStruct((ROWS, D), jnp.float32),
    compiler_params=pltpu.CompilerParams(use_tc_tiling_on_sc=False),
    scratch_types=[
        pltpu.VMEM((BPW,), jnp.int32),
        pltpu.VMEM((CHUNK, D), jnp.float32),
        pltpu.VMEM((CHUNK, D), jnp.float32),
        pltpu.SemaphoreType.DMA,
        pltpu.SemaphoreType.DMA,
        pltpu.SemaphoreType.DMA,
        pltpu.SemaphoreType.DMA,
    ],
)
def _gather(tok_hbm, idx_hbm, out_hbm, idx_v, buf0, buf1,
            semg0, semg1, semw0, semw1):
    _gather_body(tok_hbm, idx_hbm, out_hbm, idx_v, buf0, buf1,
                 semg0, semg1, semw0, semw1)


def _proj_body(emb_ref, w_ref, pos_ref, b_ref, out_ref):
    t = pl.program_id(0)
    e_bf = emb_ref[0].astype(jnp.bfloat16)
    w_bf = w_ref[...].astype(jnp.bfloat16)
    pos_bf = pos_ref[pl.ds(t, 1), :].astype(jnp.bfloat16)
    r = lax.dot_general(w_bf, e_bf, (((0,), (1,)), ((), ())),
                        preferred_element_type=jnp.float32)
    r2 = lax.dot_general(w_bf, pos_bf, (((0,), (1,)), ((), ())),
                         preferred_element_type=jnp.float32)
    out_ref[0] = r + r2 + b_ref[...]


def _project(emb3, w, pos, b_col):
    return pl.pallas_call(
        _proj_body,
        grid=(T, B // BB),
        in_specs=[
            pl.BlockSpec((1, BB, D), lambda t, j: (t, j, 0)),
            pl.BlockSpec((D, N), lambda t, j: (0, 0)),
            pl.BlockSpec((T, D), lambda t, j: (0, 0)),
            pl.BlockSpec((N, 1), lambda t, j: (0, 0)),
        ],
        out_specs=pl.BlockSpec((1, N, BB), lambda t, j: (t, 0, j)),
        out_shape=jax.ShapeDtypeStruct((T, N, B), jnp.float32),
    )(emb3, w, pos, b_col)


def kernel(idx, token_embed_table, pos_embed_table, lm_head_w, lm_head_b):
    idx = idx[:, -T:]
    idx_t = jnp.transpose(idx).reshape(ROWS).astype(jnp.int32)
    emb = _gather(token_embed_table, idx_t)
    emb3 = emb.reshape(T, B, D)
    out_t = _project(emb3, lm_head_w, pos_embed_table,
                     lm_head_b.reshape(N, 1))
    return jnp.transpose(out_t, (2, 0, 1))
